# baseline (speedup 1.0000x reference)
import functools
import math

import jax
import jax.numpy as jnp
from jax import lax
from jax.experimental import pallas as pl
from jax.experimental.pallas import tpu as pltpu
from jax.experimental.pallas import tpu_sc as plsc

_B, _N, _M = 4, 2048, 1024
_K0, _K1, _KA = 32, 32, 16
_NEG = -3e38
_NWORKERS = 32


def _gather_chunk(b_per_w, D):
    max_rows = max(8, 16384 // D)
    ch = 8
    for c in range(8, max_rows + 1, 8):
        if b_per_w % c == 0:
            ch = c
    return ch


@functools.cache
def _make_sc_gather(V, D, B_total):
    assert D % 16 == 0 and B_total % (8 * _NWORKERS) == 0
    b_per_w = B_total // _NWORKERS
    CH = _gather_chunk(b_per_w, D)
    nsteps = b_per_w // CH
    mesh = plsc.VectorSubcoreMesh(core_axis_name="c", subcore_axis_name="s")

    @functools.partial(
        pl.kernel,
        mesh=mesh,
        out_type=jax.ShapeDtypeStruct((B_total, D), jnp.float32),
        scratch_types=[
            pltpu.VMEM((CH,), jnp.int32),
            pltpu.VMEM((CH, D), jnp.float32),
            pltpu.SemaphoreType.DMA,
        ],
    )
    def gk(table_hbm, idx_hbm, out_hbm, idx_v, rows_v, sem):
        wid = lax.axis_index("s") * 2 + lax.axis_index("c")
        base = wid * b_per_w

        def step(i, carry):
            off = base + i * CH
            pltpu.sync_copy(idx_hbm.at[pl.ds(off, CH)], idx_v)
            pltpu.async_copy(table_hbm.at[idx_v], rows_v, sem).wait()
            pltpu.sync_copy(rows_v, out_hbm.at[pl.ds(off, CH)])
            return carry

        lax.fori_loop(0, nsteps, step, 0)

    return gk


def _sc_gather(table, idx):
    V, D = table.shape
    (B_total,) = idx.shape
    return _make_sc_gather(V, D, B_total)(table, idx)


def _knn_body(q_ref, b_ref, iota_ref, idx_ref, K, Nb, with_w, *maybe_w):
    q = q_ref[0]
    base = b_ref[0]
    T = q.shape[0]
    dot = lax.dot_general(q, base, (((1,), (1,)), ((), ())),
                          preferred_element_type=jnp.float32)
    sqq = jnp.sum(q * q, axis=1, keepdims=True)
    sqb = jnp.sum(base * base, axis=1)[None, :]
    neg = (2.0 * dot - sqq) - sqb
    iota_col = iota_ref[...]
    cols = []
    vals = []
    for _ in range(K):
        m = jnp.max(neg, axis=1, keepdims=True)
        eq = neg == m
        eqf = jnp.where(eq, 1.0, 0.0)
        neg = jnp.where(eq, _NEG, neg)
        ajf = lax.dot_general(eqf, iota_col, (((1,), (0,)), ((), ())),
                              precision=lax.Precision.HIGHEST,
                              preferred_element_type=jnp.float32)[:, :1]
        cols.append(ajf)
        vals.append(m)
    idx = jnp.minimum(jnp.concatenate(cols, axis=1), float(Nb - 1)
                      ).astype(jnp.int32)
    idx_ref[0] = idx + pl.program_id(0) * Nb
    if with_w:
        w_ref = maybe_w[0]
        d2 = jnp.maximum(-jnp.concatenate(vals, axis=1), 0.0)
        w = 1.0 / (d2 + 1e-8)
        w_ref[0] = w / jnp.sum(w, axis=1, keepdims=True)


@functools.cache
def _make_knn(B, Nq, Nb, C, K, with_w, T):
    def wrapped2(q_ref, b_ref, iota_ref, *out_refs):
        _knn_body(q_ref, b_ref, iota_ref, out_refs[0], K, Nb, with_w,
                  *out_refs[1:])

    out_shape = [jax.ShapeDtypeStruct((B, Nq, K), jnp.int32)]
    out_specs = [pl.BlockSpec((1, T, K), lambda b, i: (b, i, 0))]
    if with_w:
        out_shape.append(jax.ShapeDtypeStruct((B, Nq, K), jnp.float32))
        out_specs.append(pl.BlockSpec((1, T, K), lambda b, i: (b, i, 0)))
    return pl.pallas_call(
        wrapped2,
        grid=(B, Nq // T),
        in_specs=[
            pl.BlockSpec((1, T, C), lambda b, i: (b, i, 0)),
            pl.BlockSpec((1, Nb, C), lambda b, i: (b, 0, 0)),
            pl.BlockSpec((Nb, 8), lambda b, i: (0, 0)),
        ],
        out_specs=out_specs,
        out_shape=out_shape,
    )


def _iota_col(Nb):
    return jnp.zeros((Nb, 8), jnp.float32).at[:, 0].set(
        jnp.arange(Nb, dtype=jnp.float32))


def _knn(q_rows, base_rows, K, with_w=False, T=256):
    B, Nq, C = q_rows.shape
    Nb = base_rows.shape[1]
    res = _make_knn(B, Nq, Nb, C, K, with_w, T)(
        q_rows, base_rows, _iota_col(Nb))
    return res if with_w else res[0]


def _lrelu(v):
    return jnp.where(v > 0, v, 0.2 * v)


@functools.cache
def _make_edge1_fused(B, N, K, O, T):
    def body(q_ref, b_ref, w1_ref, w2_ref, out_ref):
        q = q_ref[0]
        base = b_ref[0]
        dot = lax.dot_general(q, base, (((1,), (1,)), ((), ())),
                              preferred_element_type=jnp.float32)
        sqq = jnp.sum(q * q, axis=1, keepdims=True)
        sqb = jnp.sum(base * base, axis=1)[None, :]
        neg = (2.0 * dot - sqq) - sqb
        w1 = w1_ref[...]
        w2 = w2_ref[...]
        yc = lax.dot_general(q, w2, (((1,), (0,)), ((), ())),
                             preferred_element_type=jnp.float32)
        acc = jnp.full((T, O), _NEG, jnp.float32)
        for _ in range(K):
            m = jnp.max(neg, axis=1, keepdims=True)
            eq = neg == m
            eqf = jnp.where(eq, 1.0, 0.0)
            neg = jnp.where(eq, _NEG, neg)
            nbr = lax.dot_general(eqf, base, (((1,), (0,)), ((), ())),
                                  precision=lax.Precision.HIGHEST,
                                  preferred_element_type=jnp.float32)
            y = lax.dot_general(nbr - q, w1, (((1,), (0,)), ((), ())),
                                preferred_element_type=jnp.float32) + yc
            acc = jnp.maximum(acc, _lrelu(y))
        out_ref[0] = jnp.concatenate(
            [acc, jnp.zeros((T, 128 - O), jnp.float32)], axis=1)

    return pl.pallas_call(
        body,
        grid=(B, N // T),
        in_specs=[
            pl.BlockSpec((1, T, 8), lambda b, i: (b, i, 0)),
            pl.BlockSpec((1, N, 8), lambda b, i: (b, 0, 0)),
            pl.BlockSpec((8, O), lambda b, i: (0, 0)),
            pl.BlockSpec((8, O), lambda b, i: (0, 0)),
        ],
        out_specs=pl.BlockSpec((1, T, 128), lambda b, i: (b, i, 0)),
        out_shape=jax.ShapeDtypeStruct((B, N, 128), jnp.float32),
    )


@functools.cache
def _make_edge(B, N, K, Cp, O, concat_ctr, Cc, T):
    def body(nbr_ref, ctr_ref, w1_ref, w2_ref, out_ref):
        nbr = nbr_ref[0]
        ctr = ctr_ref[0]
        w1 = w1_ref[...]
        w2 = w2_ref[...]
        diff = nbr.reshape(T, K, Cp) - ctr[:, None, :]
        y1 = lax.dot_general(diff.reshape(T * K, Cp), w1,
                             (((1,), (0,)), ((), ())),
                             preferred_element_type=jnp.float32)
        y2 = lax.dot_general(ctr, w2, (((1,), (0,)), ((), ())),
                             preferred_element_type=jnp.float32)
        y = _lrelu(y1.reshape(T, K, O) + y2[:, None, :])
        red = jnp.max(y, axis=1)
        if concat_ctr:
            out_ref[0] = jnp.concatenate([ctr[:, :Cc], red], axis=1)
        else:
            out_ref[0] = jnp.concatenate(
                [red, jnp.zeros((T, Cc), jnp.float32)], axis=1)

    out_c = Cc + O
    return pl.pallas_call(
        body,
        grid=(B, N // T),
        in_specs=[
            pl.BlockSpec((1, T * K, Cp), lambda b, i: (b, i, 0)),
            pl.BlockSpec((1, T, Cp), lambda b, i: (b, i, 0)),
            pl.BlockSpec((Cp, O), lambda b, i: (0, 0)),
            pl.BlockSpec((Cp, O), lambda b, i: (0, 0)),
        ],
        out_specs=pl.BlockSpec((1, T, out_c), lambda b, i: (b, i, 0)),
        out_shape=jax.ShapeDtypeStruct((B, N, out_c), jnp.float32),
    )


def _edge_combine(nbr_rows, ctr_rows, W, Cin, concat_ctr, T=128):
    B, N, Cp = ctr_rows.shape
    K = nbr_rows.shape[1] // N
    O = W.shape[0]
    Cc = Cin if concat_ctr else 128 - O
    w1 = jnp.zeros((Cp, O), jnp.float32).at[:Cin].set(W[:, :Cin].T)
    w2 = jnp.zeros((Cp, O), jnp.float32).at[:Cin].set(W[:, Cin:].T)
    return _make_edge(B, N, K, Cp, O, concat_ctr, Cc, T)(
        nbr_rows, ctr_rows, w1, w2)


@functools.cache
def _make_qkv(B, N, C, T):
    def body(x_ref, w_ref, q_ref, kv_ref):
        y = lax.dot_general(x_ref[0], w_ref[...], (((1,), (0,)), ((), ())),
                            preferred_element_type=jnp.float32)
        q_ref[0] = y[:, :C]
        kv_ref[0] = y[:, C:]

    return pl.pallas_call(
        body,
        grid=(B, N // T),
        in_specs=[
            pl.BlockSpec((1, T, C), lambda b, i: (b, i, 0)),
            pl.BlockSpec((C, 3 * C), lambda b, i: (0, 0)),
        ],
        out_specs=[
            pl.BlockSpec((1, T, C), lambda b, i: (b, i, 0)),
            pl.BlockSpec((1, T, 2 * C), lambda b, i: (b, i, 0)),
        ],
        out_shape=[
            jax.ShapeDtypeStruct((B, N, C), jnp.float32),
            jax.ShapeDtypeStruct((B, N, 2 * C), jnp.float32),
        ],
    )


@functools.cache
def _make_att_combine(B, N, K, C, T):
    scale = 1.0 / math.sqrt(float(C))

    def body(x_ref, q_ref, kv_ref, wf_ref, out_ref):
        x = x_ref[0]
        q = q_ref[0]
        kv = kv_ref[0].reshape(T, K, 2 * C)
        kn = kv[:, :, :C]
        vn = kv[:, :, C:]
        logits = jnp.sum(q[:, None, :] * kn, axis=2) * scale
        m = jnp.max(logits, axis=1, keepdims=True)
        e = jnp.exp(logits - m)
        p = e / jnp.sum(e, axis=1, keepdims=True)
        agg = jnp.sum(p[:, :, None] * vn, axis=1)
        o = x + agg
        f = lax.dot_general(o, wf_ref[...], (((1,), (0,)), ((), ())),
                            preferred_element_type=jnp.float32)
        out_ref[0] = o + _lrelu(f)

    return pl.pallas_call(
        body,
        grid=(B, N // T),
        in_specs=[
            pl.BlockSpec((1, T, C), lambda b, i: (b, i, 0)),
            pl.BlockSpec((1, T, C), lambda b, i: (b, i, 0)),
            pl.BlockSpec((1, T * K, 2 * C), lambda b, i: (b, i, 0)),
            pl.BlockSpec((C, C), lambda b, i: (0, 0)),
        ],
        out_specs=pl.BlockSpec((1, T, C), lambda b, i: (b, i, 0)),
        out_shape=jax.ShapeDtypeStruct((B, N, C), jnp.float32),
    )


def _n2p_attention(x_rows, Wq, Wk, Wv, Wf, K, T=128):
    B, N, C = x_rows.shape
    idx = _knn(x_rows, x_rows, K, T=128)
    w_qkv = jnp.concatenate([Wq, Wk, Wv], axis=0).T
    q, kv = _make_qkv(B, N, C, 256)(x_rows, w_qkv)
    kv_g = _sc_gather(kv.reshape(B * N, 2 * C), idx.reshape(-1))
    out = _make_att_combine(B, N, K, C, T)(
        x_rows, q, kv_g.reshape(B, N * K, 2 * C), Wf.T)
    return out


@functools.cache
def _make_downsample(B, N, M, C, C2, T, JT):
    def body(x_ref, z_ref, w_ref, xd_ref, zd_ref):
        x = x_ref[0]
        z = z_ref[0]
        s = lax.dot_general(x, w_ref[...], (((1,), (0,)), ((), ())),
                            preferred_element_type=jnp.float32)
        st = s.reshape(1, N)
        col = lax.broadcasted_iota(jnp.int32, (1, N), 1)
        rank = jnp.zeros((1, N), jnp.float32)
        for j0 in range(0, N, JT):
            sj = s[j0:j0 + JT]
            rowi = lax.broadcasted_iota(jnp.int32, (JT, 1), 0) + j0
            gt = (sj > st).astype(jnp.float32)
            tie = jnp.logical_and(sj == st, rowi < col).astype(jnp.float32)
            rank = rank + jnp.sum(gt + tie, axis=0, keepdims=True)
        r0 = pl.program_id(1) * T
        rows = (lax.broadcasted_iota(jnp.int32, (T, 1), 0) + r0
                ).astype(jnp.float32)
        onehot = (rank == rows).astype(jnp.float32)
        xd_ref[0] = lax.dot_general(onehot, x, (((1,), (0,)), ((), ())),
                                    precision=lax.Precision.HIGHEST,
                                    preferred_element_type=jnp.float32)
        zd_ref[0] = lax.dot_general(onehot, z, (((1,), (0,)), ((), ())),
                                    precision=lax.Precision.HIGHEST,
                                    preferred_element_type=jnp.float32)

    return pl.pallas_call(
        body,
        grid=(B, M // T),
        in_specs=[
            pl.BlockSpec((1, N, C), lambda b, i: (b, 0, 0)),
            pl.BlockSpec((1, N, C2), lambda b, i: (b, 0, 0)),
            pl.BlockSpec((C, 1), lambda b, i: (0, 0)),
        ],
        out_specs=[
            pl.BlockSpec((1, T, C), lambda b, i: (b, i, 0)),
            pl.BlockSpec((1, T, C2), lambda b, i: (b, i, 0)),
        ],
        out_shape=[
            jax.ShapeDtypeStruct((B, M, C), jnp.float32),
            jax.ShapeDtypeStruct((B, M, C2), jnp.float32),
        ],
    )


@functools.cache
def _make_upsample(B, N, C, T):
    def body(xs_ref, g_ref, w_ref, wu1_ref, wu2_ref, out_ref):
        xs = xs_ref[0]
        g = g_ref[0].reshape(T, 3, C)
        w = w_ref[0]
        interp = jnp.sum(w[:, :, None] * g, axis=1)
        y = (lax.dot_general(xs, wu1_ref[...], (((1,), (0,)), ((), ())),
                             preferred_element_type=jnp.float32)
             + lax.dot_general(interp, wu2_ref[...], (((1,), (0,)), ((), ())),
                               preferred_element_type=jnp.float32))
        out_ref[0] = _lrelu(y)

    return pl.pallas_call(
        body,
        grid=(B, N // T),
        in_specs=[
            pl.BlockSpec((1, T, C), lambda b, i: (b, i, 0)),
            pl.BlockSpec((1, T * 3, C), lambda b, i: (b, i, 0)),
            pl.BlockSpec((1, T, 3), lambda b, i: (b, i, 0)),
            pl.BlockSpec((C, C), lambda b, i: (0, 0)),
            pl.BlockSpec((C, C), lambda b, i: (0, 0)),
        ],
        out_specs=pl.BlockSpec((1, T, C), lambda b, i: (b, i, 0)),
        out_shape=jax.ShapeDtypeStruct((B, N, C), jnp.float32),
    )


@functools.cache
def _make_head_stats(B, N, C, O, OT):
    def body(x_ref, w_ref, out_ref):
        x = x_ref[0]
        outs_mx = []
        outs_av = []
        for k in range(O // OT):
            wk = w_ref[...][:, k * OT:(k + 1) * OT]
            y = _lrelu(lax.dot_general(x, wk, (((1,), (0,)), ((), ())),
                                       preferred_element_type=jnp.float32))
            outs_mx.append(jnp.max(y, axis=0, keepdims=True))
            outs_av.append(jnp.sum(y, axis=0, keepdims=True) * (1.0 / N))
        mx = jnp.concatenate(outs_mx, axis=1)
        av = jnp.concatenate(outs_av, axis=1)
        out_ref[0] = jnp.concatenate([mx, av], axis=0)

    return pl.pallas_call(
        body,
        grid=(B,),
        in_specs=[
            pl.BlockSpec((1, N, C), lambda b: (b, 0, 0)),
            pl.BlockSpec((C, O), lambda b: (0, 0)),
        ],
        out_specs=pl.BlockSpec((1, 2, O), lambda b: (b, 0, 0)),
        out_shape=jax.ShapeDtypeStruct((B, 2, O), jnp.float32),
    )


@functools.cache
def _make_head_bias(B, O):
    def body(g_ref, cid_ref, w1_ref, wg_ref, out_ref):
        g = g_ref[0]
        cidv = cid_ref[0]
        cid = _lrelu(lax.dot_general(cidv, w1_ref[...],
                                     (((1,), (0,)), ((), ())),
                                     preferred_element_type=jnp.float32))
        gv = jnp.concatenate([g[0:1], g[1:2], cid], axis=1)
        out_ref[0] = lax.dot_general(gv, wg_ref[...], (((1,), (0,)), ((), ())),
                                     preferred_element_type=jnp.float32)

    return pl.pallas_call(
        body,
        grid=(B,),
        in_specs=[
            pl.BlockSpec((1, 2, 1024), lambda b: (b, 0, 0)),
            pl.BlockSpec((1, 1, 16), lambda b: (b, 0, 0)),
            pl.BlockSpec((16, 64), lambda b: (0, 0)),
            pl.BlockSpec((2112, O), lambda b: (0, 0)),
        ],
        out_specs=pl.BlockSpec((1, 1, O), lambda b: (b, 0, 0)),
        out_shape=jax.ShapeDtypeStruct((B, 1, O), jnp.float32),
    )


@functools.cache
def _make_head_final(B, N, C, H1, H2, OP, T):
    def body(x_ref, bias_ref, w2_ref, w3_ref, w4_ref, out_ref):
        x = x_ref[0]
        h = _lrelu(lax.dot_general(x, w2_ref[...], (((1,), (0,)), ((), ())),
                                   preferred_element_type=jnp.float32)
                   + bias_ref[0])
        h = _lrelu(lax.dot_general(h, w3_ref[...], (((1,), (0,)), ((), ())),
                                   preferred_element_type=jnp.float32))
        out_ref[0] = lax.dot_general(h, w4_ref[...], (((1,), (0,)), ((), ())),
                                     preferred_element_type=jnp.float32)

    return pl.pallas_call(
        body,
        grid=(B, N // T),
        in_specs=[
            pl.BlockSpec((1, T, C), lambda b, i: (b, i, 0)),
            pl.BlockSpec((1, 1, H1), lambda b, i: (b, 0, 0)),
            pl.BlockSpec((C, H1), lambda b, i: (0, 0)),
            pl.BlockSpec((H1, H2), lambda b, i: (0, 0)),
            pl.BlockSpec((H2, OP), lambda b, i: (0, 0)),
        ],
        out_specs=pl.BlockSpec((1, T, OP), lambda b, i: (b, i, 0)),
        out_shape=jax.ShapeDtypeStruct((B, N, OP), jnp.float32),
    )


def kernel(x, category_id, emb0_W, emb1_W, att0_Wq, att0_Wk, att0_Wv,
           att0_Wf, att1_Wq, att1_Wk, att1_Wv, att1_Wf, att2_Wq, att2_Wk,
           att2_Wv, att2_Wf, ds_w, us_W, conv_W, conv1_W, conv2_W, conv3_W,
           conv4_W):
    B, _, N = x.shape
    M = _M

    xt = jnp.transpose(x, (0, 2, 1))
    xt8 = jnp.concatenate(
        [xt, jnp.zeros((B, N, 5), jnp.float32)], axis=2)

    w1 = jnp.zeros((8, 64), jnp.float32).at[:3].set(emb0_W[:, :3].T)
    w2 = jnp.zeros((8, 64), jnp.float32).at[:3].set(emb0_W[:, 3:].T)
    x0 = _make_edge1_fused(B, N, _K0, 64, 128)(xt8, xt8, w1, w2)

    idx1 = _knn(x0, x0, _K1, T=128)
    nbr1 = _sc_gather(x0.reshape(B * N, 128), idx1.reshape(-1))
    xf = _edge_combine(nbr1.reshape(B, N * _K1, 128), x0, emb1_W, 64,
                       concat_ctr=True, T=128)

    xf = _n2p_attention(xf, att0_Wq, att0_Wk, att0_Wv, att0_Wf, _KA)

    xd, xyzd8 = _make_downsample(B, N, M, 128, 8, 256, 256)(
        xf, xt8, ds_w.reshape(128, 1))

    xd = _n2p_attention(xd, att1_Wq, att1_Wk, att1_Wv, att1_Wf, _KA)

    idx3, w3 = _knn(xt8, xyzd8, 3, with_w=True, T=256)
    g = _sc_gather(xd.reshape(B * M, 128), idx3.reshape(-1))
    xu = _make_upsample(B, N, 128, 256)(
        xf, g.reshape(B, N * 3, 128), w3, us_W[:, :128].T, us_W[:, 128:].T)

    x_tmp = _n2p_attention(xu, att2_Wq, att2_Wk, att2_Wv, att2_Wf, _KA)

    gstats = _make_head_stats(B, N, 128, 1024, 256)(x_tmp, conv_W.T)
    bias = _make_head_bias(B, 1024)(
        gstats, jnp.transpose(category_id, (0, 2, 1)), conv1_W.T,
        conv2_W[:, :2112].T)
    w4p = jnp.zeros((256, 64), jnp.float32).at[:, :50].set(conv4_W.T)
    out_rows = _make_head_final(B, N, 128, 1024, 256, 64, 256)(
        x_tmp, bias, conv2_W[:, 2112:].T, conv3_W.T, w4p)
    return jnp.transpose(out_rows[:, :, :50], (0, 2, 1))


# speedup vs baseline: 2.7578x; 2.7578x over previous
import functools
import math

import jax
import jax.numpy as jnp
from jax import lax
from jax.experimental import pallas as pl
from jax.experimental.pallas import tpu as pltpu
from jax.experimental.pallas import tpu_sc as plsc

_B, _N, _M = 4, 2048, 1024
_K0, _K1, _KA = 32, 32, 16
_NEG = -3e38
_NWORKERS = 32


def _gather_chunk(b_per_w, D):
    max_rows = max(8, 16384 // D)
    ch = 8
    for c in range(8, max_rows + 1, 8):
        if b_per_w % c == 0:
            ch = c
    return ch


@functools.cache
def _make_sc_gather(V, D, B_total):
    assert D % 16 == 0 and B_total % (8 * _NWORKERS) == 0
    b_per_w = B_total // _NWORKERS
    CH = _gather_chunk(b_per_w, D)
    nsteps = b_per_w // CH
    mesh = plsc.VectorSubcoreMesh(core_axis_name="c", subcore_axis_name="s")

    @functools.partial(
        pl.kernel,
        mesh=mesh,
        out_type=jax.ShapeDtypeStruct((B_total, D), jnp.float32),
        scratch_types=[
            pltpu.VMEM((CH,), jnp.int32),
            pltpu.VMEM((CH, D), jnp.float32),
            pltpu.SemaphoreType.DMA,
        ],
    )
    def gk(table_hbm, idx_hbm, out_hbm, idx_v, rows_v, sem):
        wid = lax.axis_index("s") * 2 + lax.axis_index("c")
        base = wid * b_per_w

        def step(i, carry):
            off = base + i * CH
            pltpu.sync_copy(idx_hbm.at[pl.ds(off, CH)], idx_v)
            pltpu.async_copy(table_hbm.at[idx_v], rows_v, sem).wait()
            pltpu.sync_copy(rows_v, out_hbm.at[pl.ds(off, CH)])
            return carry

        lax.fori_loop(0, nsteps, step, 0)

    return gk


def _sc_gather(table, idx):
    V, D = table.shape
    (B_total,) = idx.shape
    return _make_sc_gather(V, D, B_total)(table, idx)


def _knn_body(q_ref, b_ref, idx_ref, K, Nb, with_w, *maybe_w):
    q = q_ref[0]
    base = b_ref[0]
    T = q.shape[0]
    dot = lax.dot_general(q, base, (((1,), (1,)), ((), ())),
                          preferred_element_type=jnp.float32)
    sqq = jnp.sum(q * q, axis=1, keepdims=True)
    sqb = jnp.sum(base * base, axis=1)[None, :]
    neg = (2.0 * dot - sqq) - sqb
    iota = lax.broadcasted_iota(jnp.int32, (T, Nb), 1)
    cols = []
    vals = []
    for _ in range(K):
        m = jnp.max(neg, axis=1, keepdims=True)
        eq = neg == m
        cand = jnp.where(eq, iota, Nb)
        aj = jnp.min(cand, axis=1, keepdims=True)
        neg = jnp.where(eq, _NEG, neg)
        cols.append(aj)
        vals.append(m)
    idx = jnp.concatenate(cols, axis=1)
    idx_ref[0] = idx + pl.program_id(0) * Nb
    if with_w:
        w_ref = maybe_w[0]
        d2 = jnp.maximum(-jnp.concatenate(vals, axis=1), 0.0)
        w = 1.0 / (d2 + 1e-8)
        w_ref[0] = w / jnp.sum(w, axis=1, keepdims=True)


@functools.cache
def _make_knn(B, Nq, Nb, C, K, with_w, T):
    def wrapped2(q_ref, b_ref, *out_refs):
        _knn_body(q_ref, b_ref, out_refs[0], K, Nb, with_w,
                  *out_refs[1:])

    out_shape = [jax.ShapeDtypeStruct((B, Nq, K), jnp.int32)]
    out_specs = [pl.BlockSpec((1, T, K), lambda b, i: (b, i, 0))]
    if with_w:
        out_shape.append(jax.ShapeDtypeStruct((B, Nq, K), jnp.float32))
        out_specs.append(pl.BlockSpec((1, T, K), lambda b, i: (b, i, 0)))
    return pl.pallas_call(
        wrapped2,
        grid=(B, Nq // T),
        in_specs=[
            pl.BlockSpec((1, T, C), lambda b, i: (b, i, 0)),
            pl.BlockSpec((1, Nb, C), lambda b, i: (b, 0, 0)),
        ],
        out_specs=out_specs,
        out_shape=out_shape,
    )


def _knn(q_rows, base_rows, K, with_w=False, T=256):
    B, Nq, C = q_rows.shape
    Nb = base_rows.shape[1]
    res = _make_knn(B, Nq, Nb, C, K, with_w, T)(q_rows, base_rows)
    return res if with_w else res[0]


def _lrelu(v):
    return jnp.where(v > 0, v, 0.2 * v)


@functools.cache
def _make_edge(B, N, K, Cp, O, concat_ctr, Cc, T):
    def body(nbr_ref, ctr_ref, w1_ref, w2_ref, out_ref):
        nbr = nbr_ref[0]
        ctr = ctr_ref[0]
        w1 = w1_ref[...]
        w2 = w2_ref[...]
        diff = nbr.reshape(T, K, Cp) - ctr[:, None, :]
        y1 = lax.dot_general(diff.reshape(T * K, Cp), w1,
                             (((1,), (0,)), ((), ())),
                             preferred_element_type=jnp.float32)
        y2 = lax.dot_general(ctr, w2, (((1,), (0,)), ((), ())),
                             preferred_element_type=jnp.float32)
        y = _lrelu(y1.reshape(T, K, O) + y2[:, None, :])
        red = jnp.max(y, axis=1)
        if concat_ctr:
            out_ref[0] = jnp.concatenate([ctr[:, :Cc], red], axis=1)
        else:
            out_ref[0] = jnp.concatenate(
                [red, jnp.zeros((T, Cc), jnp.float32)], axis=1)

    out_c = Cc + O
    return pl.pallas_call(
        body,
        grid=(B, N // T),
        in_specs=[
            pl.BlockSpec((1, T * K, Cp), lambda b, i: (b, i, 0)),
            pl.BlockSpec((1, T, Cp), lambda b, i: (b, i, 0)),
            pl.BlockSpec((Cp, O), lambda b, i: (0, 0)),
            pl.BlockSpec((Cp, O), lambda b, i: (0, 0)),
        ],
        out_specs=pl.BlockSpec((1, T, out_c), lambda b, i: (b, i, 0)),
        out_shape=jax.ShapeDtypeStruct((B, N, out_c), jnp.float32),
    )


def _edge_combine(nbr_rows, ctr_rows, W, Cin, concat_ctr, T=128):
    B, N, Cp = ctr_rows.shape
    K = nbr_rows.shape[1] // N
    O = W.shape[0]
    Cc = Cin if concat_ctr else 128 - O
    w1 = jnp.zeros((Cp, O), jnp.float32).at[:Cin].set(W[:, :Cin].T)
    w2 = jnp.zeros((Cp, O), jnp.float32).at[:Cin].set(W[:, Cin:].T)
    return _make_edge(B, N, K, Cp, O, concat_ctr, Cc, T)(
        nbr_rows, ctr_rows, w1, w2)


@functools.cache
def _make_qkv(B, N, C, T):
    def body(x_ref, w_ref, q_ref, kv_ref):
        y = lax.dot_general(x_ref[0], w_ref[...], (((1,), (0,)), ((), ())),
                            preferred_element_type=jnp.float32)
        q_ref[0] = y[:, :C]
        kv_ref[0] = y[:, C:]

    return pl.pallas_call(
        body,
        grid=(B, N // T),
        in_specs=[
            pl.BlockSpec((1, T, C), lambda b, i: (b, i, 0)),
            pl.BlockSpec((C, 3 * C), lambda b, i: (0, 0)),
        ],
        out_specs=[
            pl.BlockSpec((1, T, C), lambda b, i: (b, i, 0)),
            pl.BlockSpec((1, T, 2 * C), lambda b, i: (b, i, 0)),
        ],
        out_shape=[
            jax.ShapeDtypeStruct((B, N, C), jnp.float32),
            jax.ShapeDtypeStruct((B, N, 2 * C), jnp.float32),
        ],
    )


@functools.cache
def _make_att_combine(B, N, K, C, T):
    scale = 1.0 / math.sqrt(float(C))

    def body(x_ref, q_ref, kv_ref, wf_ref, out_ref):
        x = x_ref[0]
        q = q_ref[0]
        kv = kv_ref[0].reshape(T, K, 2 * C)
        kn = kv[:, :, :C]
        vn = kv[:, :, C:]
        logits = jnp.sum(q[:, None, :] * kn, axis=2) * scale
        m = jnp.max(logits, axis=1, keepdims=True)
        e = jnp.exp(logits - m)
        p = e / jnp.sum(e, axis=1, keepdims=True)
        agg = jnp.sum(p[:, :, None] * vn, axis=1)
        o = x + agg
        f = lax.dot_general(o, wf_ref[...], (((1,), (0,)), ((), ())),
                            preferred_element_type=jnp.float32)
        out_ref[0] = o + _lrelu(f)

    return pl.pallas_call(
        body,
        grid=(B, N // T),
        in_specs=[
            pl.BlockSpec((1, T, C), lambda b, i: (b, i, 0)),
            pl.BlockSpec((1, T, C), lambda b, i: (b, i, 0)),
            pl.BlockSpec((1, T * K, 2 * C), lambda b, i: (b, i, 0)),
            pl.BlockSpec((C, C), lambda b, i: (0, 0)),
        ],
        out_specs=pl.BlockSpec((1, T, C), lambda b, i: (b, i, 0)),
        out_shape=jax.ShapeDtypeStruct((B, N, C), jnp.float32),
    )


def _n2p_attention(x_rows, Wq, Wk, Wv, Wf, K, T=128):
    B, N, C = x_rows.shape
    idx = _knn(x_rows, x_rows, K, T=128)
    w_qkv = jnp.concatenate([Wq, Wk, Wv], axis=0).T
    q, kv = _make_qkv(B, N, C, 256)(x_rows, w_qkv)
    kv_g = _sc_gather(kv.reshape(B * N, 2 * C), idx.reshape(-1))
    out = _make_att_combine(B, N, K, C, T)(
        x_rows, q, kv_g.reshape(B, N * K, 2 * C), Wf.T)
    return out


@functools.cache
def _make_downsample(B, N, M, C, C2, T, JT):
    def body(x_ref, z_ref, w_ref, xd_ref, zd_ref):
        x = x_ref[0]
        z = z_ref[0]
        s = lax.dot_general(x, w_ref[...], (((1,), (0,)), ((), ())),
                            preferred_element_type=jnp.float32)
        st = s.reshape(1, N)
        col = lax.broadcasted_iota(jnp.int32, (1, N), 1)
        rank = jnp.zeros((1, N), jnp.float32)
        for j0 in range(0, N, JT):
            sj = s[j0:j0 + JT]
            rowi = lax.broadcasted_iota(jnp.int32, (JT, 1), 0) + j0
            gt = (sj > st).astype(jnp.float32)
            tie = jnp.logical_and(sj == st, rowi < col).astype(jnp.float32)
            rank = rank + jnp.sum(gt + tie, axis=0, keepdims=True)
        r0 = pl.program_id(1) * T
        rows = (lax.broadcasted_iota(jnp.int32, (T, 1), 0) + r0
                ).astype(jnp.float32)
        onehot = (rank == rows).astype(jnp.float32)
        xd_ref[0] = lax.dot_general(onehot, x, (((1,), (0,)), ((), ())),
                                    precision=lax.Precision.HIGHEST,
                                    preferred_element_type=jnp.float32)
        zd_ref[0] = lax.dot_general(onehot, z, (((1,), (0,)), ((), ())),
                                    precision=lax.Precision.HIGHEST,
                                    preferred_element_type=jnp.float32)

    return pl.pallas_call(
        body,
        grid=(B, M // T),
        in_specs=[
            pl.BlockSpec((1, N, C), lambda b, i: (b, 0, 0)),
            pl.BlockSpec((1, N, C2), lambda b, i: (b, 0, 0)),
            pl.BlockSpec((C, 1), lambda b, i: (0, 0)),
        ],
        out_specs=[
            pl.BlockSpec((1, T, C), lambda b, i: (b, i, 0)),
            pl.BlockSpec((1, T, C2), lambda b, i: (b, i, 0)),
        ],
        out_shape=[
            jax.ShapeDtypeStruct((B, M, C), jnp.float32),
            jax.ShapeDtypeStruct((B, M, C2), jnp.float32),
        ],
    )


@functools.cache
def _make_upsample(B, N, C, T):
    def body(xs_ref, g_ref, w_ref, wu1_ref, wu2_ref, out_ref):
        xs = xs_ref[0]
        g = g_ref[0].reshape(T, 3, C)
        w = w_ref[0]
        interp = jnp.sum(w[:, :, None] * g, axis=1)
        y = (lax.dot_general(xs, wu1_ref[...], (((1,), (0,)), ((), ())),
                             preferred_element_type=jnp.float32)
             + lax.dot_general(interp, wu2_ref[...], (((1,), (0,)), ((), ())),
                               preferred_element_type=jnp.float32))
        out_ref[0] = _lrelu(y)

    return pl.pallas_call(
        body,
        grid=(B, N // T),
        in_specs=[
            pl.BlockSpec((1, T, C), lambda b, i: (b, i, 0)),
            pl.BlockSpec((1, T * 3, C), lambda b, i: (b, i, 0)),
            pl.BlockSpec((1, T, 3), lambda b, i: (b, i, 0)),
            pl.BlockSpec((C, C), lambda b, i: (0, 0)),
            pl.BlockSpec((C, C), lambda b, i: (0, 0)),
        ],
        out_specs=pl.BlockSpec((1, T, C), lambda b, i: (b, i, 0)),
        out_shape=jax.ShapeDtypeStruct((B, N, C), jnp.float32),
    )


@functools.cache
def _make_head_stats(B, N, C, O, OT):
    def body(x_ref, w_ref, out_ref):
        x = x_ref[0]
        outs_mx = []
        outs_av = []
        for k in range(O // OT):
            wk = w_ref[...][:, k * OT:(k + 1) * OT]
            y = _lrelu(lax.dot_general(x, wk, (((1,), (0,)), ((), ())),
                                       preferred_element_type=jnp.float32))
            outs_mx.append(jnp.max(y, axis=0, keepdims=True))
            outs_av.append(jnp.sum(y, axis=0, keepdims=True) * (1.0 / N))
        mx = jnp.concatenate(outs_mx, axis=1)
        av = jnp.concatenate(outs_av, axis=1)
        out_ref[0] = jnp.concatenate([mx, av], axis=0)

    return pl.pallas_call(
        body,
        grid=(B,),
        in_specs=[
            pl.BlockSpec((1, N, C), lambda b: (b, 0, 0)),
            pl.BlockSpec((C, O), lambda b: (0, 0)),
        ],
        out_specs=pl.BlockSpec((1, 2, O), lambda b: (b, 0, 0)),
        out_shape=jax.ShapeDtypeStruct((B, 2, O), jnp.float32),
    )


@functools.cache
def _make_head_bias(B, O):
    def body(g_ref, cid_ref, w1_ref, wg_ref, out_ref):
        g = g_ref[0]
        cidv = cid_ref[0]
        cid = _lrelu(lax.dot_general(cidv, w1_ref[...],
                                     (((1,), (0,)), ((), ())),
                                     preferred_element_type=jnp.float32))
        gv = jnp.concatenate([g[0:1], g[1:2], cid], axis=1)
        out_ref[0] = lax.dot_general(gv, wg_ref[...], (((1,), (0,)), ((), ())),
                                     preferred_element_type=jnp.float32)

    return pl.pallas_call(
        body,
        grid=(B,),
        in_specs=[
            pl.BlockSpec((1, 2, 1024), lambda b: (b, 0, 0)),
            pl.BlockSpec((1, 1, 16), lambda b: (b, 0, 0)),
            pl.BlockSpec((16, 64), lambda b: (0, 0)),
            pl.BlockSpec((2112, O), lambda b: (0, 0)),
        ],
        out_specs=pl.BlockSpec((1, 1, O), lambda b: (b, 0, 0)),
        out_shape=jax.ShapeDtypeStruct((B, 1, O), jnp.float32),
    )


@functools.cache
def _make_head_final(B, N, C, H1, H2, OP, T):
    def body(x_ref, bias_ref, w2_ref, w3_ref, w4_ref, out_ref):
        x = x_ref[0]
        h = _lrelu(lax.dot_general(x, w2_ref[...], (((1,), (0,)), ((), ())),
                                   preferred_element_type=jnp.float32)
                   + bias_ref[0])
        h = _lrelu(lax.dot_general(h, w3_ref[...], (((1,), (0,)), ((), ())),
                                   preferred_element_type=jnp.float32))
        out_ref[0] = lax.dot_general(h, w4_ref[...], (((1,), (0,)), ((), ())),
                                     preferred_element_type=jnp.float32)

    return pl.pallas_call(
        body,
        grid=(B, N // T),
        in_specs=[
            pl.BlockSpec((1, T, C), lambda b, i: (b, i, 0)),
            pl.BlockSpec((1, 1, H1), lambda b, i: (b, 0, 0)),
            pl.BlockSpec((C, H1), lambda b, i: (0, 0)),
            pl.BlockSpec((H1, H2), lambda b, i: (0, 0)),
            pl.BlockSpec((H2, OP), lambda b, i: (0, 0)),
        ],
        out_specs=pl.BlockSpec((1, T, OP), lambda b, i: (b, i, 0)),
        out_shape=jax.ShapeDtypeStruct((B, N, OP), jnp.float32),
    )


def kernel(x, category_id, emb0_W, emb1_W, att0_Wq, att0_Wk, att0_Wv,
           att0_Wf, att1_Wq, att1_Wk, att1_Wv, att1_Wf, att2_Wq, att2_Wk,
           att2_Wv, att2_Wf, ds_w, us_W, conv_W, conv1_W, conv2_W, conv3_W,
           conv4_W):
    B, _, N = x.shape
    M = _M

    xt = jnp.transpose(x, (0, 2, 1))
    xt8 = jnp.concatenate(
        [xt, jnp.zeros((B, N, 5), jnp.float32)], axis=2)
    xt128 = jnp.concatenate(
        [xt, jnp.zeros((B, N, 125), jnp.float32)], axis=2)

    idx0 = _knn(xt8, xt8, _K0, T=256)
    nbr0 = _sc_gather(xt128.reshape(B * N, 128), idx0.reshape(-1))
    x0 = _edge_combine(nbr0.reshape(B, N * _K0, 128), xt128, emb0_W, 3,
                       concat_ctr=False, T=128)

    idx1 = _knn(x0, x0, _K1, T=128)
    nbr1 = _sc_gather(x0.reshape(B * N, 128), idx1.reshape(-1))
    xf = _edge_combine(nbr1.reshape(B, N * _K1, 128), x0, emb1_W, 64,
                       concat_ctr=True, T=128)

    xf = _n2p_attention(xf, att0_Wq, att0_Wk, att0_Wv, att0_Wf, _KA)

    xd, xyzd8 = _make_downsample(B, N, M, 128, 8, 256, 256)(
        xf, xt8, ds_w.reshape(128, 1))

    xd = _n2p_attention(xd, att1_Wq, att1_Wk, att1_Wv, att1_Wf, _KA)

    idx3, w3 = _knn(xt8, xyzd8, 3, with_w=True, T=256)
    g = _sc_gather(xd.reshape(B * M, 128), idx3.reshape(-1))
    xu = _make_upsample(B, N, 128, 256)(
        xf, g.reshape(B, N * 3, 128), w3, us_W[:, :128].T, us_W[:, 128:].T)

    x_tmp = _n2p_attention(xu, att2_Wq, att2_Wk, att2_Wv, att2_Wf, _KA)

    gstats = _make_head_stats(B, N, 128, 1024, 256)(x_tmp, conv_W.T)
    bias = _make_head_bias(B, 1024)(
        gstats, jnp.transpose(category_id, (0, 2, 1)), conv1_W.T,
        conv2_W[:, :2112].T)
    w4p = jnp.zeros((256, 64), jnp.float32).at[:, :50].set(conv4_W.T)
    out_rows = _make_head_final(B, N, 128, 1024, 256, 64, 256)(
        x_tmp, bias, conv2_W[:, 2112:].T, conv3_W.T, w4p)
    return jnp.transpose(out_rows[:, :, :50], (0, 2, 1))


# speedup vs baseline: 2.8570x; 1.0360x over previous
import functools
import math

import jax
import jax.numpy as jnp
from jax import lax
from jax.experimental import pallas as pl
from jax.experimental.pallas import tpu as pltpu
from jax.experimental.pallas import tpu_sc as plsc

_B, _N, _M = 4, 2048, 1024
_K0, _K1, _KA = 32, 32, 16
_NEG = -3e38
_NWORKERS = 32


def _gather_chunk(b_per_w, D):
    max_rows = max(8, 16384 // D)
    ch = 8
    for c in range(8, max_rows + 1, 8):
        if b_per_w % c == 0:
            ch = c
    return ch


@functools.cache
def _make_sc_gather(V, D, B_total):
    assert D % 16 == 0 and B_total % (8 * _NWORKERS) == 0
    b_per_w = B_total // _NWORKERS
    CH = _gather_chunk(b_per_w, D)
    nsteps = b_per_w // CH
    mesh = plsc.VectorSubcoreMesh(core_axis_name="c", subcore_axis_name="s")

    @functools.partial(
        pl.kernel,
        mesh=mesh,
        out_type=jax.ShapeDtypeStruct((B_total, D), jnp.float32),
        scratch_types=[
            pltpu.VMEM((CH,), jnp.int32),
            pltpu.VMEM((CH, D), jnp.float32),
            pltpu.SemaphoreType.DMA,
        ],
    )
    def gk(table_hbm, idx_hbm, out_hbm, idx_v, rows_v, sem):
        wid = lax.axis_index("s") * 2 + lax.axis_index("c")
        base = wid * b_per_w

        def step(i, carry):
            off = base + i * CH
            pltpu.sync_copy(idx_hbm.at[pl.ds(off, CH)], idx_v)
            pltpu.async_copy(table_hbm.at[idx_v], rows_v, sem).wait()
            pltpu.sync_copy(rows_v, out_hbm.at[pl.ds(off, CH)])
            return carry

        lax.fori_loop(0, nsteps, step, 0)

    return gk


def _sc_gather(table, idx):
    V, D = table.shape
    (B_total,) = idx.shape
    return _make_sc_gather(V, D, B_total)(table, idx)


def _knn_body(q_ref, b_ref, idx_ref, K, Nb, with_w, *maybe_w):
    q = q_ref[0]
    base = b_ref[0]
    T = q.shape[0]
    dot = lax.dot_general(q, base, (((1,), (1,)), ((), ())),
                          preferred_element_type=jnp.float32)
    sqq = jnp.sum(q * q, axis=1, keepdims=True)
    sqb = jnp.sum(base * base, axis=1)[None, :]
    neg = (2.0 * dot - sqq) - sqb
    iota = lax.broadcasted_iota(jnp.int32, (T, Nb), 1)
    if with_w:
        cols = []
        vals = []
        for _ in range(K):
            m = jnp.max(neg, axis=1, keepdims=True)
            eq = neg == m
            cand = jnp.where(eq, iota, Nb)
            aj = jnp.min(cand, axis=1, keepdims=True)
            neg = jnp.where(eq, _NEG, neg)
            cols.append(aj)
            vals.append(m)
        idx = jnp.concatenate(cols, axis=1)
        idx_ref[0] = idx + pl.program_id(0) * Nb
        w_ref = maybe_w[0]
        d2 = jnp.maximum(-jnp.concatenate(vals, axis=1), 0.0)
        w = 1.0 / (d2 + 1e-8)
        w_ref[0] = w / jnp.sum(w, axis=1, keepdims=True)
    else:
        bits = lax.bitcast_convert_type(neg, jnp.int32)
        key = bits ^ (lax.shift_right_arithmetic(bits, 31) & 0x7FFFFFFF)
        kk = (key & jnp.int32(-2048)) | (2047 - iota)
        cols = []
        for _ in range(K):
            m = jnp.max(kk, axis=1, keepdims=True)
            kk = jnp.where(kk == m, jnp.int32(-2147483648), kk)
            cols.append(2047 - (m & 2047))
        idx = jnp.concatenate(cols, axis=1)
        idx_ref[0] = idx + pl.program_id(0) * Nb


@functools.cache
def _make_knn(B, Nq, Nb, C, K, with_w, T):
    def wrapped2(q_ref, b_ref, *out_refs):
        _knn_body(q_ref, b_ref, out_refs[0], K, Nb, with_w,
                  *out_refs[1:])

    out_shape = [jax.ShapeDtypeStruct((B, Nq, K), jnp.int32)]
    out_specs = [pl.BlockSpec((1, T, K), lambda b, i: (b, i, 0))]
    if with_w:
        out_shape.append(jax.ShapeDtypeStruct((B, Nq, K), jnp.float32))
        out_specs.append(pl.BlockSpec((1, T, K), lambda b, i: (b, i, 0)))
    return pl.pallas_call(
        wrapped2,
        grid=(B, Nq // T),
        in_specs=[
            pl.BlockSpec((1, T, C), lambda b, i: (b, i, 0)),
            pl.BlockSpec((1, Nb, C), lambda b, i: (b, 0, 0)),
        ],
        out_specs=out_specs,
        out_shape=out_shape,
    )


def _knn(q_rows, base_rows, K, with_w=False, T=256):
    B, Nq, C = q_rows.shape
    Nb = base_rows.shape[1]
    res = _make_knn(B, Nq, Nb, C, K, with_w, T)(q_rows, base_rows)
    return res if with_w else res[0]


def _lrelu(v):
    return jnp.where(v > 0, v, 0.2 * v)


@functools.cache
def _make_edge(B, N, K, Cp, O, concat_ctr, Cc, T):
    def body(nbr_ref, ctr_ref, w1_ref, w2_ref, out_ref):
        nbr = nbr_ref[0]
        ctr = ctr_ref[0]
        w1 = w1_ref[...]
        w2 = w2_ref[...]
        diff = nbr.reshape(T, K, Cp) - ctr[:, None, :]
        y1 = lax.dot_general(diff.reshape(T * K, Cp), w1,
                             (((1,), (0,)), ((), ())),
                             preferred_element_type=jnp.float32)
        y2 = lax.dot_general(ctr, w2, (((1,), (0,)), ((), ())),
                             preferred_element_type=jnp.float32)
        y = _lrelu(y1.reshape(T, K, O) + y2[:, None, :])
        red = jnp.max(y, axis=1)
        if concat_ctr:
            out_ref[0] = jnp.concatenate([ctr[:, :Cc], red], axis=1)
        else:
            out_ref[0] = jnp.concatenate(
                [red, jnp.zeros((T, Cc), jnp.float32)], axis=1)

    out_c = Cc + O
    return pl.pallas_call(
        body,
        grid=(B, N // T),
        in_specs=[
            pl.BlockSpec((1, T * K, Cp), lambda b, i: (b, i, 0)),
            pl.BlockSpec((1, T, Cp), lambda b, i: (b, i, 0)),
            pl.BlockSpec((Cp, O), lambda b, i: (0, 0)),
            pl.BlockSpec((Cp, O), lambda b, i: (0, 0)),
        ],
        out_specs=pl.BlockSpec((1, T, out_c), lambda b, i: (b, i, 0)),
        out_shape=jax.ShapeDtypeStruct((B, N, out_c), jnp.float32),
    )


def _edge_combine(nbr_rows, ctr_rows, W, Cin, concat_ctr, T=128):
    B, N, Cp = ctr_rows.shape
    K = nbr_rows.shape[1] // N
    O = W.shape[0]
    Cc = Cin if concat_ctr else 128 - O
    w1 = jnp.zeros((Cp, O), jnp.float32).at[:Cin].set(W[:, :Cin].T)
    w2 = jnp.zeros((Cp, O), jnp.float32).at[:Cin].set(W[:, Cin:].T)
    return _make_edge(B, N, K, Cp, O, concat_ctr, Cc, T)(
        nbr_rows, ctr_rows, w1, w2)


@functools.cache
def _make_qkv(B, N, C, T):
    def body(x_ref, w_ref, q_ref, kv_ref):
        y = lax.dot_general(x_ref[0], w_ref[...], (((1,), (0,)), ((), ())),
                            preferred_element_type=jnp.float32)
        q_ref[0] = y[:, :C]
        kv_ref[0] = y[:, C:]

    return pl.pallas_call(
        body,
        grid=(B, N // T),
        in_specs=[
            pl.BlockSpec((1, T, C), lambda b, i: (b, i, 0)),
            pl.BlockSpec((C, 3 * C), lambda b, i: (0, 0)),
        ],
        out_specs=[
            pl.BlockSpec((1, T, C), lambda b, i: (b, i, 0)),
            pl.BlockSpec((1, T, 2 * C), lambda b, i: (b, i, 0)),
        ],
        out_shape=[
            jax.ShapeDtypeStruct((B, N, C), jnp.float32),
            jax.ShapeDtypeStruct((B, N, 2 * C), jnp.float32),
        ],
    )


@functools.cache
def _make_att_combine(B, N, K, C, T):
    scale = 1.0 / math.sqrt(float(C))

    def body(x_ref, q_ref, kv_ref, wf_ref, out_ref):
        x = x_ref[0]
        q = q_ref[0]
        kv = kv_ref[0].reshape(T, K, 2 * C)
        kn = kv[:, :, :C]
        vn = kv[:, :, C:]
        logits = jnp.sum(q[:, None, :] * kn, axis=2) * scale
        m = jnp.max(logits, axis=1, keepdims=True)
        e = jnp.exp(logits - m)
        p = e / jnp.sum(e, axis=1, keepdims=True)
        agg = jnp.sum(p[:, :, None] * vn, axis=1)
        o = x + agg
        f = lax.dot_general(o, wf_ref[...], (((1,), (0,)), ((), ())),
                            preferred_element_type=jnp.float32)
        out_ref[0] = o + _lrelu(f)

    return pl.pallas_call(
        body,
        grid=(B, N // T),
        in_specs=[
            pl.BlockSpec((1, T, C), lambda b, i: (b, i, 0)),
            pl.BlockSpec((1, T, C), lambda b, i: (b, i, 0)),
            pl.BlockSpec((1, T * K, 2 * C), lambda b, i: (b, i, 0)),
            pl.BlockSpec((C, C), lambda b, i: (0, 0)),
        ],
        out_specs=pl.BlockSpec((1, T, C), lambda b, i: (b, i, 0)),
        out_shape=jax.ShapeDtypeStruct((B, N, C), jnp.float32),
    )


def _n2p_attention(x_rows, Wq, Wk, Wv, Wf, K, T=128):
    B, N, C = x_rows.shape
    idx = _knn(x_rows, x_rows, K, T=128)
    w_qkv = jnp.concatenate([Wq, Wk, Wv], axis=0).T
    q, kv = _make_qkv(B, N, C, 256)(x_rows, w_qkv)
    kv_g = _sc_gather(kv.reshape(B * N, 2 * C), idx.reshape(-1))
    out = _make_att_combine(B, N, K, C, T)(
        x_rows, q, kv_g.reshape(B, N * K, 2 * C), Wf.T)
    return out


@functools.cache
def _make_downsample(B, N, M, C, C2, T, JT):
    def body(x_ref, z_ref, w_ref, xd_ref, zd_ref):
        x = x_ref[0]
        z = z_ref[0]
        s = lax.dot_general(x, w_ref[...], (((1,), (0,)), ((), ())),
                            preferred_element_type=jnp.float32)
        st = s.reshape(1, N)
        col = lax.broadcasted_iota(jnp.int32, (1, N), 1)
        rank = jnp.zeros((1, N), jnp.float32)
        for j0 in range(0, N, JT):
            sj = s[j0:j0 + JT]
            rowi = lax.broadcasted_iota(jnp.int32, (JT, 1), 0) + j0
            gt = (sj > st).astype(jnp.float32)
            tie = jnp.logical_and(sj == st, rowi < col).astype(jnp.float32)
            rank = rank + jnp.sum(gt + tie, axis=0, keepdims=True)
        r0 = pl.program_id(1) * T
        rows = (lax.broadcasted_iota(jnp.int32, (T, 1), 0) + r0
                ).astype(jnp.float32)
        onehot = (rank == rows).astype(jnp.float32)
        xd_ref[0] = lax.dot_general(onehot, x, (((1,), (0,)), ((), ())),
                                    precision=lax.Precision.HIGHEST,
                                    preferred_element_type=jnp.float32)
        zd_ref[0] = lax.dot_general(onehot, z, (((1,), (0,)), ((), ())),
                                    precision=lax.Precision.HIGHEST,
                                    preferred_element_type=jnp.float32)

    return pl.pallas_call(
        body,
        grid=(B, M // T),
        in_specs=[
            pl.BlockSpec((1, N, C), lambda b, i: (b, 0, 0)),
            pl.BlockSpec((1, N, C2), lambda b, i: (b, 0, 0)),
            pl.BlockSpec((C, 1), lambda b, i: (0, 0)),
        ],
        out_specs=[
            pl.BlockSpec((1, T, C), lambda b, i: (b, i, 0)),
            pl.BlockSpec((1, T, C2), lambda b, i: (b, i, 0)),
        ],
        out_shape=[
            jax.ShapeDtypeStruct((B, M, C), jnp.float32),
            jax.ShapeDtypeStruct((B, M, C2), jnp.float32),
        ],
    )


@functools.cache
def _make_upsample(B, N, C, T):
    def body(xs_ref, g_ref, w_ref, wu1_ref, wu2_ref, out_ref):
        xs = xs_ref[0]
        g = g_ref[0].reshape(T, 3, C)
        w = w_ref[0]
        interp = jnp.sum(w[:, :, None] * g, axis=1)
        y = (lax.dot_general(xs, wu1_ref[...], (((1,), (0,)), ((), ())),
                             preferred_element_type=jnp.float32)
             + lax.dot_general(interp, wu2_ref[...], (((1,), (0,)), ((), ())),
                               preferred_element_type=jnp.float32))
        out_ref[0] = _lrelu(y)

    return pl.pallas_call(
        body,
        grid=(B, N // T),
        in_specs=[
            pl.BlockSpec((1, T, C), lambda b, i: (b, i, 0)),
            pl.BlockSpec((1, T * 3, C), lambda b, i: (b, i, 0)),
            pl.BlockSpec((1, T, 3), lambda b, i: (b, i, 0)),
            pl.BlockSpec((C, C), lambda b, i: (0, 0)),
            pl.BlockSpec((C, C), lambda b, i: (0, 0)),
        ],
        out_specs=pl.BlockSpec((1, T, C), lambda b, i: (b, i, 0)),
        out_shape=jax.ShapeDtypeStruct((B, N, C), jnp.float32),
    )


@functools.cache
def _make_head_stats(B, N, C, O, OT):
    def body(x_ref, w_ref, out_ref):
        x = x_ref[0]
        outs_mx = []
        outs_av = []
        for k in range(O // OT):
            wk = w_ref[...][:, k * OT:(k + 1) * OT]
            y = _lrelu(lax.dot_general(x, wk, (((1,), (0,)), ((), ())),
                                       preferred_element_type=jnp.float32))
            outs_mx.append(jnp.max(y, axis=0, keepdims=True))
            outs_av.append(jnp.sum(y, axis=0, keepdims=True) * (1.0 / N))
        mx = jnp.concatenate(outs_mx, axis=1)
        av = jnp.concatenate(outs_av, axis=1)
        out_ref[0] = jnp.concatenate([mx, av], axis=0)

    return pl.pallas_call(
        body,
        grid=(B,),
        in_specs=[
            pl.BlockSpec((1, N, C), lambda b: (b, 0, 0)),
            pl.BlockSpec((C, O), lambda b: (0, 0)),
        ],
        out_specs=pl.BlockSpec((1, 2, O), lambda b: (b, 0, 0)),
        out_shape=jax.ShapeDtypeStruct((B, 2, O), jnp.float32),
    )


@functools.cache
def _make_head_bias(B, O):
    def body(g_ref, cid_ref, w1_ref, wg_ref, out_ref):
        g = g_ref[0]
        cidv = cid_ref[0]
        cid = _lrelu(lax.dot_general(cidv, w1_ref[...],
                                     (((1,), (0,)), ((), ())),
                                     preferred_element_type=jnp.float32))
        gv = jnp.concatenate([g[0:1], g[1:2], cid], axis=1)
        out_ref[0] = lax.dot_general(gv, wg_ref[...], (((1,), (0,)), ((), ())),
                                     preferred_element_type=jnp.float32)

    return pl.pallas_call(
        body,
        grid=(B,),
        in_specs=[
            pl.BlockSpec((1, 2, 1024), lambda b: (b, 0, 0)),
            pl.BlockSpec((1, 1, 16), lambda b: (b, 0, 0)),
            pl.BlockSpec((16, 64), lambda b: (0, 0)),
            pl.BlockSpec((2112, O), lambda b: (0, 0)),
        ],
        out_specs=pl.BlockSpec((1, 1, O), lambda b: (b, 0, 0)),
        out_shape=jax.ShapeDtypeStruct((B, 1, O), jnp.float32),
    )


@functools.cache
def _make_head_final(B, N, C, H1, H2, OP, T):
    def body(x_ref, bias_ref, w2_ref, w3_ref, w4_ref, out_ref):
        x = x_ref[0]
        h = _lrelu(lax.dot_general(x, w2_ref[...], (((1,), (0,)), ((), ())),
                                   preferred_element_type=jnp.float32)
                   + bias_ref[0])
        h = _lrelu(lax.dot_general(h, w3_ref[...], (((1,), (0,)), ((), ())),
                                   preferred_element_type=jnp.float32))
        out_ref[0] = lax.dot_general(h, w4_ref[...], (((1,), (0,)), ((), ())),
                                     preferred_element_type=jnp.float32)

    return pl.pallas_call(
        body,
        grid=(B, N // T),
        in_specs=[
            pl.BlockSpec((1, T, C), lambda b, i: (b, i, 0)),
            pl.BlockSpec((1, 1, H1), lambda b, i: (b, 0, 0)),
            pl.BlockSpec((C, H1), lambda b, i: (0, 0)),
            pl.BlockSpec((H1, H2), lambda b, i: (0, 0)),
            pl.BlockSpec((H2, OP), lambda b, i: (0, 0)),
        ],
        out_specs=pl.BlockSpec((1, T, OP), lambda b, i: (b, i, 0)),
        out_shape=jax.ShapeDtypeStruct((B, N, OP), jnp.float32),
    )


def kernel(x, category_id, emb0_W, emb1_W, att0_Wq, att0_Wk, att0_Wv,
           att0_Wf, att1_Wq, att1_Wk, att1_Wv, att1_Wf, att2_Wq, att2_Wk,
           att2_Wv, att2_Wf, ds_w, us_W, conv_W, conv1_W, conv2_W, conv3_W,
           conv4_W):
    B, _, N = x.shape
    M = _M

    xt = jnp.transpose(x, (0, 2, 1))
    xt8 = jnp.concatenate(
        [xt, jnp.zeros((B, N, 5), jnp.float32)], axis=2)
    xt128 = jnp.concatenate(
        [xt, jnp.zeros((B, N, 125), jnp.float32)], axis=2)

    idx0 = _knn(xt8, xt8, _K0, T=256)
    nbr0 = _sc_gather(xt128.reshape(B * N, 128), idx0.reshape(-1))
    x0 = _edge_combine(nbr0.reshape(B, N * _K0, 128), xt128, emb0_W, 3,
                       concat_ctr=False, T=128)

    idx1 = _knn(x0, x0, _K1, T=128)
    nbr1 = _sc_gather(x0.reshape(B * N, 128), idx1.reshape(-1))
    xf = _edge_combine(nbr1.reshape(B, N * _K1, 128), x0, emb1_W, 64,
                       concat_ctr=True, T=128)

    xf = _n2p_attention(xf, att0_Wq, att0_Wk, att0_Wv, att0_Wf, _KA)

    xd, xyzd8 = _make_downsample(B, N, M, 128, 8, 256, 256)(
        xf, xt8, ds_w.reshape(128, 1))

    xd = _n2p_attention(xd, att1_Wq, att1_Wk, att1_Wv, att1_Wf, _KA)

    idx3, w3 = _knn(xt8, xyzd8, 3, with_w=True, T=256)
    g = _sc_gather(xd.reshape(B * M, 128), idx3.reshape(-1))
    xu = _make_upsample(B, N, 128, 256)(
        xf, g.reshape(B, N * 3, 128), w3, us_W[:, :128].T, us_W[:, 128:].T)

    x_tmp = _n2p_attention(xu, att2_Wq, att2_Wk, att2_Wv, att2_Wf, _KA)

    gstats = _make_head_stats(B, N, 128, 1024, 256)(x_tmp, conv_W.T)
    bias = _make_head_bias(B, 1024)(
        gstats, jnp.transpose(category_id, (0, 2, 1)), conv1_W.T,
        conv2_W[:, :2112].T)
    w4p = jnp.zeros((256, 64), jnp.float32).at[:, :50].set(conv4_W.T)
    out_rows = _make_head_final(B, N, 128, 1024, 256, 64, 256)(
        x_tmp, bias, conv2_W[:, 2112:].T, conv3_W.T, w4p)
    return jnp.transpose(out_rows[:, :, :50], (0, 2, 1))
